# baseline (speedup 1.0000x reference)
import jax
import jax.numpy as jnp
from jax.experimental import pallas as pl

_K = 32


def _body(u_ref, i_ref, o_ref):
    p = u_ref[...] * i_ref[...]
    cols = p.shape[1]
    sel = (
        jax.lax.broadcasted_iota(jnp.int32, (cols, cols // _K), 0) // _K
        == jax.lax.broadcasted_iota(jnp.int32, (cols, cols // _K), 1)
    ).astype(jnp.float32)
    o_ref[...] = jnp.dot(p, sel, preferred_element_type=jnp.float32)


def kernel(gu, gi):
    gu = jnp.squeeze(gu)
    gi = jnp.squeeze(gi)
    n, k = gu.shape
    pack = 1024 // k
    n4 = n // pack
    gu2 = gu.reshape(n4, pack * k)
    gi2 = gi.reshape(n4, pack * k)
    bm = 2000
    grid = n4 // bm
    out = pl.pallas_call(
        _body,
        grid=(grid,),
        in_specs=[
            pl.BlockSpec((bm, pack * k), lambda i: (i, 0)),
            pl.BlockSpec((bm, pack * k), lambda i: (i, 0)),
        ],
        out_specs=pl.BlockSpec((bm, pack), lambda i: (i, 0)),
        out_shape=jax.ShapeDtypeStruct((n4, pack), jnp.float32),
    )(gu2, gi2)
    return out.reshape(n)


# speedup vs baseline: 13.1023x; 13.1023x over previous
import jax
import jax.numpy as jnp
from jax.experimental import pallas as pl


def _body(u_ref, i_ref, o_ref):
    o_ref[...] = jnp.sum(u_ref[...] * i_ref[...], axis=0)


def kernel(gu, gi):
    gu = jnp.squeeze(gu)
    gi = jnp.squeeze(gi)
    n, k = gu.shape
    ut = gu.T
    it = gi.T
    bn = 65536
    grid = pl.cdiv(n, bn)
    return pl.pallas_call(
        _body,
        grid=(grid,),
        in_specs=[
            pl.BlockSpec((k, bn), lambda i: (0, i)),
            pl.BlockSpec((k, bn), lambda i: (0, i)),
        ],
        out_specs=pl.BlockSpec((bn,), lambda i: (i,)),
        out_shape=jax.ShapeDtypeStruct((n,), jnp.float32),
    )(ut, it)
